# baseline (speedup 1.0000x reference)
import functools

import jax
import jax.numpy as jnp
from jax.experimental import pallas as pl
from jax.experimental.pallas import tpu as pltpu

K = 8192
D = 128
BETA = 0.25
TEMP = 0.1
DECAY = 0.9

BR = 512
TK = 512
N_ROWS = 4096
NB = N_ROWS // BR
NT = K // TK


def _vq_kernel(z_ref, emb_ref, ema_ref,
               zq_ref, commit_ref, perp_ref, ent_ref, emamin_ref,
               e_scr, en_scr, embbf_scr, psum_scr, counts_scr, commit_scr):
    i = pl.program_id(0)

    @pl.when(i == 0)
    def _init():
        psum_scr[...] = jnp.zeros_like(psum_scr)
        counts_scr[...] = jnp.zeros_like(counts_scr)
        commit_scr[...] = jnp.zeros_like(commit_scr)
        emb = emb_ref[...]
        en_scr[...] = emb / jnp.maximum(
            jnp.sqrt(jnp.sum(emb * emb, axis=1, keepdims=True)), 1e-12)
        embbf_scr[...] = emb.astype(jnp.bfloat16)

    zb = z_ref[...]
    zn = zb / jnp.maximum(
        jnp.sqrt(jnp.sum(zb * zb, axis=1, keepdims=True)), 1e-12)

    cols = jax.lax.broadcasted_iota(jnp.int32, (BR, TK), 1)
    ones_tk = jnp.ones((TK, 128), dtype=jnp.bfloat16)
    ones_br = jnp.ones((BR, 1), dtype=jnp.bfloat16)
    wrow = jax.lax.broadcasted_iota(jnp.int32, (TK, 128), 0)
    wcol = jax.lax.broadcasted_iota(jnp.int32, (TK, 128), 1)
    ch = wrow // 256
    cm = (wrow // 16) % 16
    cl = wrow % 16
    wint = jnp.where(wcol == 0, ch,
           jnp.where(wcol == 1, cm,
           jnp.where(wcol == 2, cl,
           jnp.where(wcol == 3, ch * cm,
           jnp.where(wcol == 4, ch * cl,
           jnp.where(wcol == 5, cm * cm,
           jnp.where(wcol == 6, cm * cl,
           jnp.where(wcol == 7, cl * cl,
           jnp.where(wcol == 8, 1, 0)))))))))
    wext = wint.astype(jnp.bfloat16)

    m = jnp.full((BR, 1), -jnp.inf, dtype=jnp.float32)
    bidx = jnp.zeros((BR, 1), dtype=jnp.int32)
    se = jnp.zeros((BR, 128), dtype=jnp.float32)
    for t in range(NT):
        en = en_scr[pl.ds(t * TK, TK), :]
        cos = jax.lax.dot_general(
            zn, en, (((1,), (1,)), ((), ())),
            preferred_element_type=jnp.float32)
        e = jnp.exp(cos * (1.0 / TEMP)).astype(jnp.bfloat16)
        e_scr[:, pl.ds(t * TK, TK)] = e
        se = se + jax.lax.dot_general(
            e, ones_tk, (((1,), (0,)), ((), ())),
            preferred_element_type=jnp.float32)
        lm = jnp.max(cos, axis=1, keepdims=True)
        eqb = (cos == lm).astype(jnp.bfloat16)
        ext = jax.lax.dot_general(
            eqb, wext, (((1,), (0,)), ((), ())),
            preferred_element_type=jnp.float32)
        a = 256.0 * ext[:, 0:1] + 16.0 * ext[:, 1:2] + ext[:, 2:3]
        b = (65536.0 * ext[:, 0:1] + 8192.0 * ext[:, 3:4]
             + 512.0 * ext[:, 4:5] + 256.0 * ext[:, 5:6]
             + 32.0 * ext[:, 6:7] + ext[:, 7:8])
        n = ext[:, 8:9]
        la = (a - jnp.sqrt(jnp.maximum(n * b - a * a, 0.0))) / n
        la_i = la.astype(jnp.int32) + t * TK
        upd = lm > m
        m = jnp.where(upd, lm, m)
        bidx = jnp.where(upd, la_i, bidx)

    rinv = (1.0 / se[:, 0:1]).astype(jnp.bfloat16)

    zq = jnp.zeros((BR, D), dtype=jnp.float32)
    for t in range(NT):
        e = e_scr[:, pl.ds(t * TK, TK)]
        psum_scr[0:1, pl.ds(t * TK, TK)] += jax.lax.dot_general(
            rinv, e, (((0,), (0,)), ((), ())),
            preferred_element_type=jnp.float32)
        onehot = (cols == bidx - t * TK).astype(jnp.bfloat16)
        counts_scr[0:1, pl.ds(t * TK, TK)] += jax.lax.dot_general(
            ones_br, onehot, (((0,), (0,)), ((), ())),
            preferred_element_type=jnp.float32)
        et = embbf_scr[pl.ds(t * TK, TK), :]
        zq = zq + jax.lax.dot_general(
            onehot, et, (((1,), (0,)), ((), ())),
            preferred_element_type=jnp.float32)

    zq_ref[...] = zq
    diff = zq - zb
    commit_scr[...] += jnp.sum(diff * diff).reshape(1, 1)

    @pl.when(i == NB - 1)
    def _finalize():
        counts = counts_scr[...]
        e_mean = counts * (1.0 / N_ROWS)
        perp = jnp.exp(-jnp.sum(e_mean * jnp.log(e_mean + 1e-8)))
        p_avg = psum_scr[...] * (1.0 / N_ROWS) + 1e-8
        ent = -jnp.sum(p_avg * jnp.log(p_avg))
        new_ema = DECAY * ema_ref[...] + (1.0 - DECAY) * e_mean
        thr = 0.0125 / K
        new_ema = jnp.where(new_ema < thr, 1.0 / K, new_ema)
        commit_ref[...] = (1.0 + BETA) / (N_ROWS * D) * commit_scr[...]
        perp_ref[...] = perp.reshape(1, 1)
        ent_ref[...] = ent.reshape(1, 1)
        emamin_ref[...] = jnp.min(new_ema).reshape(1, 1)


@functools.partial(jax.jit, static_argnames=("interpret",))
def _run(z_flat, embedding_weight, ema2d, interpret=False):
    out_shapes = (
        jax.ShapeDtypeStruct((N_ROWS, D), jnp.float32),
        jax.ShapeDtypeStruct((1, 1), jnp.float32),
        jax.ShapeDtypeStruct((1, 1), jnp.float32),
        jax.ShapeDtypeStruct((1, 1), jnp.float32),
        jax.ShapeDtypeStruct((1, 1), jnp.float32),
    )
    grid_spec = pltpu.PrefetchScalarGridSpec(
        num_scalar_prefetch=0,
        grid=(NB,),
        in_specs=[
            pl.BlockSpec((BR, D), lambda i: (i, 0)),
            pl.BlockSpec((K, D), lambda i: (0, 0)),
            pl.BlockSpec((1, K), lambda i: (0, 0)),
        ],
        out_specs=(
            pl.BlockSpec((BR, D), lambda i: (i, 0)),
            pl.BlockSpec((1, 1), lambda i: (0, 0)),
            pl.BlockSpec((1, 1), lambda i: (0, 0)),
            pl.BlockSpec((1, 1), lambda i: (0, 0)),
            pl.BlockSpec((1, 1), lambda i: (0, 0)),
        ),
        scratch_shapes=[
            pltpu.VMEM((BR, K), jnp.bfloat16),
            pltpu.VMEM((K, D), jnp.float32),
            pltpu.VMEM((K, D), jnp.bfloat16),
            pltpu.VMEM((1, K), jnp.float32),
            pltpu.VMEM((1, K), jnp.float32),
            pltpu.VMEM((1, 1), jnp.float32),
        ],
    )
    return pl.pallas_call(
        _vq_kernel,
        grid_spec=grid_spec,
        out_shape=out_shapes,
        interpret=interpret,
    )(z_flat, embedding_weight, ema2d)


def kernel(z, embedding_weight, codebook_probs_ema):
    orig_shape = z.shape
    z_flat = z.reshape(-1, D)
    ema2d = codebook_probs_ema.reshape(1, K)
    zq, commit, perp, ent, emamin = _run(z_flat, embedding_weight, ema2d)
    return (zq.reshape(orig_shape), commit[0, 0], perp[0, 0],
            ent[0, 0], emamin[0, 0])


# speedup vs baseline: 1.9946x; 1.9946x over previous
import functools

import jax
import jax.numpy as jnp
from jax.experimental import pallas as pl
from jax.experimental.pallas import tpu as pltpu

K = 8192
D = 128
BETA = 0.25
TEMP = 0.1
DECAY = 0.9

BR = 1024
TK = 512
N_ROWS = 4096
NB = N_ROWS // BR
NT = K // TK


def _vq_kernel(z_ref, emb_ref, ema_ref,
               zq_ref, commit_ref, perp_ref, ent_ref, emamin_ref,
               e_scr, en_scr, embbf_scr, psum_scr, counts_scr, commit_scr):
    i = pl.program_id(0)

    @pl.when(i == 0)
    def _init():
        psum_scr[...] = jnp.zeros_like(psum_scr)
        counts_scr[...] = jnp.zeros_like(counts_scr)
        commit_scr[...] = jnp.zeros_like(commit_scr)
        emb = emb_ref[...]
        en_scr[...] = emb / jnp.maximum(
            jnp.sqrt(jnp.sum(emb * emb, axis=1, keepdims=True)), 1e-12)
        embbf_scr[...] = emb.astype(jnp.bfloat16)

    zb = z_ref[...]
    zn = zb / jnp.maximum(
        jnp.sqrt(jnp.sum(zb * zb, axis=1, keepdims=True)), 1e-12)

    cols = jax.lax.broadcasted_iota(jnp.int32, (BR, TK), 1)
    ones_tk = jnp.ones((TK, 128), dtype=jnp.bfloat16)
    ones_br = jnp.ones((BR, 1), dtype=jnp.bfloat16)
    m = jnp.full((BR, 1), -jnp.inf, dtype=jnp.float32)
    bidx = jnp.zeros((BR, 1), dtype=jnp.int32)
    se = jnp.zeros((BR, 128), dtype=jnp.float32)
    for t in range(NT):
        en = en_scr[pl.ds(t * TK, TK), :]
        cos = jax.lax.dot_general(
            zn, en, (((1,), (1,)), ((), ())),
            preferred_element_type=jnp.float32)
        e = jnp.exp2(cos * 14.426950408889634).astype(jnp.bfloat16)
        e_scr[:, pl.ds(t * TK, TK)] = e
        se = se + jax.lax.dot_general(
            e, ones_tk, (((1,), (0,)), ((), ())),
            preferred_element_type=jnp.float32)
        lm = jnp.max(cos, axis=1, keepdims=True)
        cand = jnp.where(cos == lm, cols, K)
        la_i = jnp.min(cand, axis=1, keepdims=True) + t * TK
        upd = lm > m
        m = jnp.where(upd, lm, m)
        bidx = jnp.where(upd, la_i, bidx)

    rinv = (1.0 / se[:, 0:1]).astype(jnp.bfloat16)

    zq = jnp.zeros((BR, D), dtype=jnp.float32)
    for t in range(NT):
        e = e_scr[:, pl.ds(t * TK, TK)]
        psum_scr[0:1, pl.ds(t * TK, TK)] += jax.lax.dot_general(
            rinv, e, (((0,), (0,)), ((), ())),
            preferred_element_type=jnp.float32)
        onehot = (cols == bidx - t * TK).astype(jnp.bfloat16)
        counts_scr[0:1, pl.ds(t * TK, TK)] += jax.lax.dot_general(
            ones_br, onehot, (((0,), (0,)), ((), ())),
            preferred_element_type=jnp.float32)
        et = embbf_scr[pl.ds(t * TK, TK), :]
        zq = zq + jax.lax.dot_general(
            onehot, et, (((1,), (0,)), ((), ())),
            preferred_element_type=jnp.float32)

    zq_ref[...] = zq
    diff = zq - zb
    commit_scr[...] += jnp.sum(diff * diff).reshape(1, 1)

    @pl.when(i == NB - 1)
    def _finalize():
        counts = counts_scr[...]
        e_mean = counts * (1.0 / N_ROWS)
        perp = jnp.exp(-jnp.sum(e_mean * jnp.log(e_mean + 1e-8)))
        p_avg = psum_scr[...] * (1.0 / N_ROWS) + 1e-8
        ent = -jnp.sum(p_avg * jnp.log(p_avg))
        new_ema = DECAY * ema_ref[...] + (1.0 - DECAY) * e_mean
        thr = 0.0125 / K
        new_ema = jnp.where(new_ema < thr, 1.0 / K, new_ema)
        commit_ref[...] = (1.0 + BETA) / (N_ROWS * D) * commit_scr[...]
        perp_ref[...] = perp.reshape(1, 1)
        ent_ref[...] = ent.reshape(1, 1)
        emamin_ref[...] = jnp.min(new_ema).reshape(1, 1)


@functools.partial(jax.jit, static_argnames=("interpret",))
def _run(z_flat, embedding_weight, ema2d, interpret=False):
    out_shapes = (
        jax.ShapeDtypeStruct((N_ROWS, D), jnp.float32),
        jax.ShapeDtypeStruct((1, 1), jnp.float32),
        jax.ShapeDtypeStruct((1, 1), jnp.float32),
        jax.ShapeDtypeStruct((1, 1), jnp.float32),
        jax.ShapeDtypeStruct((1, 1), jnp.float32),
    )
    grid_spec = pltpu.PrefetchScalarGridSpec(
        num_scalar_prefetch=0,
        grid=(NB,),
        in_specs=[
            pl.BlockSpec((BR, D), lambda i: (i, 0)),
            pl.BlockSpec((K, D), lambda i: (0, 0)),
            pl.BlockSpec((1, K), lambda i: (0, 0)),
        ],
        out_specs=(
            pl.BlockSpec((BR, D), lambda i: (i, 0)),
            pl.BlockSpec((1, 1), lambda i: (0, 0)),
            pl.BlockSpec((1, 1), lambda i: (0, 0)),
            pl.BlockSpec((1, 1), lambda i: (0, 0)),
            pl.BlockSpec((1, 1), lambda i: (0, 0)),
        ),
        scratch_shapes=[
            pltpu.VMEM((BR, K), jnp.bfloat16),
            pltpu.VMEM((K, D), jnp.float32),
            pltpu.VMEM((K, D), jnp.bfloat16),
            pltpu.VMEM((1, K), jnp.float32),
            pltpu.VMEM((1, K), jnp.float32),
            pltpu.VMEM((1, 1), jnp.float32),
        ],
    )
    return pl.pallas_call(
        _vq_kernel,
        grid_spec=grid_spec,
        out_shape=out_shapes,
        interpret=interpret,
    )(z_flat, embedding_weight, ema2d)


def kernel(z, embedding_weight, codebook_probs_ema):
    orig_shape = z.shape
    z_flat = z.reshape(-1, D)
    ema2d = codebook_probs_ema.reshape(1, K)
    zq, commit, perp, ent, emamin = _run(z_flat, embedding_weight, ema2d)
    return (zq.reshape(orig_shape), commit[0, 0], perp[0, 0],
            ent[0, 0], emamin[0, 0])
